# baseline (speedup 1.0000x reference)
import functools

import jax
import jax.numpy as jnp
from jax import lax
from jax.experimental import pallas as pl
from jax.experimental.pallas import tpu as pltpu
from jax.experimental.pallas import tpu_sc as plsc

_N = 50000
_F = 3
_T = 32
_S = 32
_NSEG = 128
_SCALE = 500.0

_NLANE = 16
_HB = 4 * _S * _T
_CHN = 2048
_XTOT = _N * _F
_XLEN = _CHN * _F + 8
_SEG_PER_TILE = 4


def _sc_body(xf_hbm, vrep_hbm, cst_hbm, bnd_hbm, out_hbm,
             xb_v, vrep_v, cst_v, bnd_v, hist_v, ect_v, sem):
    wid = lax.axis_index("s") * 2 + lax.axis_index("c")

    pltpu.sync_copy(vrep_hbm, vrep_v)
    pltpu.sync_copy(cst_hbm, cst_v)
    pltpu.sync_copy(bnd_hbm, bnd_v)

    def zbody(i, c):
        for u in range(8):
            hist_v[pl.ds((i * 8 + u) * _NLANE, _NLANE)] = jnp.zeros(
                (_NLANE,), jnp.float32
            )
        return c

    lax.fori_loop(0, _HB // (_NLANE * 8), zbody, 0)

    inv_d = cst_v[pl.ds(0, _NLANE)]
    k_u = cst_v[pl.ds(_NLANE, _NLANE)]
    sd = cst_v[pl.ds(2 * _NLANE, _NLANE)]
    c0 = cst_v[pl.ds(3 * _NLANE, _NLANE)]
    vs = [
        [vrep_v[pl.ds((t * _F + c) * _NLANE, _NLANE)] for c in range(_F)]
        for t in range(_T)
    ]
    ln_iota = lax.iota(jnp.int32, _NLANE)
    bw = bnd_v[pl.ds(wid * _NLANE, _NLANE)]

    for segl in range(_SEG_PER_TILE):
        lo = jnp.sum(jnp.where(ln_iota == segl, bw, 0))
        hi = jnp.sum(jnp.where(ln_iota == segl + 1, bw, 0))
        nchunks = (hi - lo + (_CHN - 1)) >> 11

        def cbody(ci, carry, lo=lo, hi=hi, segl=segl):
            n0 = lo + ci * _CHN
            cnt = jnp.minimum(hi - n0, _CHN)
            f = n0 * _F
            a0 = pl.multiple_of(jnp.minimum(f & -8, _XTOT - _XLEN), 8)
            pltpu.sync_copy(xf_hbm.at[pl.ds(a0, _XLEN)], xb_v)
            s0 = f - a0

            def gbody(g, gc, cnt=cnt, s0=s0, segl=segl):
                ln = g * _NLANE + ln_iota
                gm = ln < cnt
                bidx = s0 + ln * _F
                x0 = plsc.load_gather(xb_v, [bidx])
                x1 = plsc.load_gather(xb_v, [bidx + 1])
                x2 = plsc.load_gather(xb_v, [bidx + 2])
                for t in range(_T):
                    nh = x0 * vs[t][0] + x1 * vs[t][1] + x2 * vs[t][2]
                    u5 = nh * inv_d + k_u
                    ji = jnp.maximum(u5, jnp.float32(1.0)).astype(jnp.int32)
                    j = ji - 2
                    zneg = jnp.float32(_SCALE) * nh - (
                        j.astype(jnp.float32) * sd + c0
                    )
                    wgt = jnp.float32(1.0) / (jnp.float32(1.0) + jnp.exp(zneg))
                    wgt = jnp.where(j >= 0, wgt, jnp.float32(0.0))
                    bin1 = (j << 5) + (segl * _S * _T + t)
                    m1 = jnp.logical_and(
                        gm, jnp.logical_and(j >= 0, j <= _S - 1)
                    )
                    plsc.addupdate_scatter(hist_v, [bin1], wgt, mask=m1)
                    m2 = jnp.logical_and(
                        gm, jnp.logical_and(j >= -1, j <= _S - 2)
                    )
                    plsc.addupdate_scatter(
                        hist_v, [bin1 + _T], jnp.float32(1.0) - wgt, mask=m2
                    )
                return gc

            lax.fori_loop(0, (cnt + (_NLANE - 1)) >> 4, gbody, 0)
            return carry

        lax.fori_loop(0, nchunks, cbody, 0)

    for segl in range(_SEG_PER_TILE):
        base = segl * _S * _T

        def rbody(s, acc, base=base):
            a0, a1 = acc
            p = base + s * _T
            a0 = a0 + hist_v[pl.ds(p, _NLANE)]
            a1 = a1 + hist_v[pl.ds(p + _NLANE, _NLANE)]
            ect_v[pl.ds(p, _NLANE)] = a0
            ect_v[pl.ds(p + _NLANE, _NLANE)] = a1
            return (a0, a1)

        z16 = jnp.zeros((_NLANE,), jnp.float32)
        lax.fori_loop(0, _S, rbody, (z16, z16))

    pltpu.sync_copy(ect_v, out_hbm.at[pl.ds(wid * _HB, _HB)])


def kernel(x, batch, v, lin):
    bnd = jnp.searchsorted(
        batch, jnp.arange(0, _NSEG + 1, dtype=batch.dtype)
    ).astype(jnp.int32)
    row = jnp.arange(_T) * _SEG_PER_TILE
    bndrep = bnd[jnp.minimum(row[:, None] + jnp.arange(_NLANE)[None, :], _NSEG)]
    vrep = jnp.repeat(v.T.reshape(-1), _NLANE)
    lin0 = lin[0]
    d = (lin[_S - 1] - lin0) / jnp.float32(_S - 1)
    inv_d = jnp.float32(1.0) / d
    cst = jnp.repeat(
        jnp.stack([
            inv_d,
            jnp.float32(2.5) - lin0 * inv_d,
            jnp.float32(_SCALE) * d,
            jnp.float32(_SCALE) * lin0,
        ]),
        _NLANE,
    )

    sc_hist = functools.partial(
        pl.kernel,
        out_type=jax.ShapeDtypeStruct((_NSEG * _S * _T,), jnp.float32),
        mesh=plsc.VectorSubcoreMesh(core_axis_name="c", subcore_axis_name="s"),
        compiler_params=pltpu.CompilerParams(needs_layout_passes=False),
        scratch_types=[
            pltpu.VMEM((_XLEN,), jnp.float32),
            pltpu.VMEM((_T * _F * _NLANE,), jnp.float32),
            pltpu.VMEM((4 * _NLANE,), jnp.float32),
            pltpu.VMEM((_T * _NLANE,), jnp.int32),
            pltpu.VMEM((_HB,), jnp.float32),
            pltpu.VMEM((_HB,), jnp.float32),
            pltpu.SemaphoreType.DMA,
        ],
    )(_sc_body)

    out = sc_hist(x.reshape(_XTOT), vrep, cst, bndrep.reshape(-1))
    return out.reshape(_NSEG, _S, _T)


# speedup vs baseline: 1.7936x; 1.7936x over previous
import functools

import jax
import jax.numpy as jnp
from jax import lax
from jax.experimental import pallas as pl
from jax.experimental.pallas import tpu as pltpu
from jax.experimental.pallas import tpu_sc as plsc

_N = 50000
_F = 3
_T = 32
_S = 32
_NSEG = 128
_SCALE = 500.0

_NB = 2048
_NP = _N
_G = (_NP + _NB - 1) // _NB
_HB = _NSEG * _S
_NLANE = 16
_WEPS = 1e-3


def _encode_kernel(x_ref, b_ref, v_ref, lin_ref, out_ref):
    x_blk = x_ref[...]
    v = v_ref[...]
    nh = lax.dot_general(
        v, x_blk, (((0,), (1,)), ((), ())), preferred_element_type=jnp.float32
    )
    lin = lin_ref[...]
    lin0 = lin[0, 0]
    d = (lin[0, _S - 1] - lin0) / jnp.float32(_S - 1)
    u = (nh - lin0) * (jnp.float32(1.0) / d)
    jf = jnp.floor(u + jnp.float32(0.5))
    jc = jnp.clip(jf, -1.0, jnp.float32(_S))
    w = jax.nn.sigmoid(_SCALE * (lin0 + jc * d - nh))
    w = jnp.clip(w, _WEPS, 1.0 - _WEPS)
    w = jnp.where(jc < 0.0, jnp.float32(1.0 - _WEPS), w)
    jb = jnp.maximum(jc, 0.0)
    seg = b_ref[0]
    idx = seg[None, :].astype(jnp.float32) * jnp.float32(_S) + jb
    nid = pl.program_id(0) * _NB + jax.lax.broadcasted_iota(
        jnp.int32, (_T, _NB), 1
    )
    dead = jnp.logical_or(jc >= jnp.float32(_S), nid >= _N)
    idx = jnp.where(dead, jnp.float32(4 * _HB), idx)
    out_ref[...] = idx + w


def _sc_hist_body(val_hbm, out_hbm, val_v, hist_v, ect_v, sem):
    t = lax.axis_index("s") * 2 + lax.axis_index("c")

    cp = pltpu.async_copy(val_hbm.at[pl.ds(t * _NP, _NP)], val_v, sem)

    def zbody(i, c):
        for u in range(8):
            hist_v[pl.ds((i * 8 + u) * _NLANE, _NLANE)] = jnp.zeros(
                (_NLANE,), jnp.float32
            )
        return c

    lax.fori_loop(0, _HB // (_NLANE * 8), zbody, 0)

    cp.wait()

    _UNROLL = 5

    def ibody(i, carry):
        for u in range(_UNROLL):
            val = val_v[pl.ds((i * _UNROLL + u) * _NLANE, _NLANE)]
            idx = val.astype(jnp.int32)
            w = val - idx.astype(jnp.float32)
            m1 = idx < _HB
            plsc.addupdate_scatter(hist_v, [idx], w, mask=m1)
            m2 = jnp.logical_and(m1, (idx & (_S - 1)) != (_S - 1))
            plsc.addupdate_scatter(hist_v, [idx + 1], 1.0 - w, mask=m2)
        return carry

    lax.fori_loop(0, _NP // (_NLANE * _UNROLL), ibody, 0)

    def rbody(b, carry):
        a0 = hist_v[pl.ds(b * _S, _NLANE)]
        a1 = hist_v[pl.ds(b * _S + _NLANE, _NLANE)]
        c0 = jnp.cumsum(a0)
        c1 = jnp.cumsum(a1) + jnp.sum(a0)
        ect_v[pl.ds(b * _S, _NLANE)] = c0
        ect_v[pl.ds(b * _S + _NLANE, _NLANE)] = c1
        return carry

    lax.fori_loop(0, _NSEG, rbody, 0)
    pltpu.sync_copy(ect_v, out_hbm.at[pl.ds(t * _HB, _HB)])


def kernel(x, batch, v, lin):
    val = pl.pallas_call(
        _encode_kernel,
        grid=(_G,),
        in_specs=[
            pl.BlockSpec((_NB, _F), lambda i: (i, 0)),
            pl.BlockSpec((1, _NB), lambda i: (0, i)),
            pl.BlockSpec((_F, _T), lambda i: (0, 0)),
            pl.BlockSpec((1, _S), lambda i: (0, 0)),
        ],
        out_specs=pl.BlockSpec((_T, _NB), lambda i: (0, i)),
        out_shape=jax.ShapeDtypeStruct((_T, _NP), jnp.float32),
    )(x, batch.reshape(1, _NP), v, lin.reshape(1, _S))

    sc_hist = functools.partial(
        pl.kernel,
        out_type=jax.ShapeDtypeStruct((_T * _HB,), jnp.float32),
        mesh=plsc.VectorSubcoreMesh(core_axis_name="c", subcore_axis_name="s"),
        compiler_params=pltpu.CompilerParams(needs_layout_passes=False),
        scratch_types=[
            pltpu.VMEM((_NP,), jnp.float32),
            pltpu.VMEM((_HB,), jnp.float32),
            pltpu.VMEM((_HB,), jnp.float32),
            pltpu.SemaphoreType.DMA,
        ],
    )(_sc_hist_body)

    ect_tbs = sc_hist(val.reshape(_T * _NP))
    return ect_tbs.reshape(_T, _NSEG, _S).transpose(1, 2, 0)


# speedup vs baseline: 1.7961x; 1.0014x over previous
import functools

import jax
import jax.numpy as jnp
from jax import lax
from jax.experimental import pallas as pl
from jax.experimental.pallas import tpu as pltpu
from jax.experimental.pallas import tpu_sc as plsc

_N = 50000
_F = 3
_T = 32
_S = 32
_NSEG = 128
_SCALE = 500.0

_NB = 2048
_NP = _N
_G = (_NP + _NB - 1) // _NB
_HB = _NSEG * _S
_NLANE = 16
_WEPS = 1e-3


def _encode_kernel(x_ref, b_ref, v_ref, lin_ref, out_ref):
    x_blk = x_ref[...]
    v = v_ref[...]
    nh = lax.dot_general(
        v, x_blk, (((0,), (1,)), ((), ())), preferred_element_type=jnp.float32
    )
    lin = lin_ref[...]
    lin0 = lin[0, 0]
    d = (lin[0, _S - 1] - lin0) / jnp.float32(_S - 1)
    u = (nh - lin0) * (jnp.float32(1.0) / d)
    jf = jnp.floor(u + jnp.float32(0.5))
    jc = jnp.clip(jf, -1.0, jnp.float32(_S))
    w = jax.nn.sigmoid(_SCALE * (lin0 + jc * d - nh))
    w = jnp.clip(w, _WEPS, 1.0 - _WEPS)
    w = jnp.where(jc < 0.0, jnp.float32(1.0 - _WEPS), w)
    jb = jnp.maximum(jc, 0.0)
    seg = b_ref[0]
    idx = seg[None, :].astype(jnp.float32) * jnp.float32(_S) + jb
    nid = pl.program_id(0) * _NB + jax.lax.broadcasted_iota(
        jnp.int32, (_T, _NB), 1
    )
    dead = jnp.logical_or(jc >= jnp.float32(_S), nid >= _N)
    idx = jnp.where(dead, jnp.float32(4 * _HB), idx)
    out_ref[...] = idx + w


def _sc_hist_body(val_hbm, out_hbm, val_v, hist_v, ect_v, sem):
    t = lax.axis_index("s") * 2 + lax.axis_index("c")

    cp = pltpu.async_copy(val_hbm.at[pl.ds(t * _NP, _NP)], val_v, sem)

    def zbody(i, c):
        for u in range(8):
            hist_v[pl.ds((i * 8 + u) * _NLANE, _NLANE)] = jnp.zeros(
                (_NLANE,), jnp.float32
            )
        return c

    lax.fori_loop(0, _HB // (_NLANE * 8), zbody, 0)

    cp.wait()

    _UNROLL = 25

    def ibody(i, carry):
        for u in range(_UNROLL):
            val = val_v[pl.ds((i * _UNROLL + u) * _NLANE, _NLANE)]
            idx = val.astype(jnp.int32)
            w = val - idx.astype(jnp.float32)
            m1 = idx < _HB
            plsc.addupdate_scatter(hist_v, [idx], w, mask=m1)
            m2 = jnp.logical_and(m1, (idx & (_S - 1)) != (_S - 1))
            plsc.addupdate_scatter(hist_v, [idx + 1], 1.0 - w, mask=m2)
        return carry

    lax.fori_loop(0, _NP // (_NLANE * _UNROLL), ibody, 0)

    def rbody(b, carry):
        a0 = hist_v[pl.ds(b * _S, _NLANE)]
        a1 = hist_v[pl.ds(b * _S + _NLANE, _NLANE)]
        c0 = jnp.cumsum(a0)
        c1 = jnp.cumsum(a1) + jnp.sum(a0)
        ect_v[pl.ds(b * _S, _NLANE)] = c0
        ect_v[pl.ds(b * _S + _NLANE, _NLANE)] = c1
        return carry

    lax.fori_loop(0, _NSEG, rbody, 0)
    pltpu.sync_copy(ect_v, out_hbm.at[pl.ds(t * _HB, _HB)])


def kernel(x, batch, v, lin):
    val = pl.pallas_call(
        _encode_kernel,
        grid=(_G,),
        in_specs=[
            pl.BlockSpec((_NB, _F), lambda i: (i, 0)),
            pl.BlockSpec((1, _NB), lambda i: (0, i)),
            pl.BlockSpec((_F, _T), lambda i: (0, 0)),
            pl.BlockSpec((1, _S), lambda i: (0, 0)),
        ],
        out_specs=pl.BlockSpec((_T, _NB), lambda i: (0, i)),
        out_shape=jax.ShapeDtypeStruct((_T, _NP), jnp.float32),
    )(x, batch.reshape(1, _NP), v, lin.reshape(1, _S))

    sc_hist = functools.partial(
        pl.kernel,
        out_type=jax.ShapeDtypeStruct((_T * _HB,), jnp.float32),
        mesh=plsc.VectorSubcoreMesh(core_axis_name="c", subcore_axis_name="s"),
        compiler_params=pltpu.CompilerParams(needs_layout_passes=False),
        scratch_types=[
            pltpu.VMEM((_NP,), jnp.float32),
            pltpu.VMEM((_HB,), jnp.float32),
            pltpu.VMEM((_HB,), jnp.float32),
            pltpu.SemaphoreType.DMA,
        ],
    )(_sc_hist_body)

    ect_tbs = sc_hist(val.reshape(_T * _NP))
    return ect_tbs.reshape(_T, _NSEG, _S).transpose(1, 2, 0)
